# baseline (speedup 1.0000x reference)
import functools

import jax
import jax.numpy as jnp
from jax.experimental import pallas as pl
from jax.experimental.pallas import tpu as pltpu


def _router_kernel(x_ref, w_ref, bs_ref, br_ref,
                   logits_ref, weights_ref, pooled_ref, *, sd, ne):
    s = pl.program_id(1)
    ns = pl.num_programs(1)
    x = x_ref[0].astype(jnp.bfloat16)

    out = jnp.dot(x, w_ref[...], preferred_element_type=jnp.float32)

    logits = out[:, sd:sd + ne] + br_ref[...]
    logits_ref[0] = logits
    m = jnp.max(logits, axis=-1, keepdims=True)
    e = jnp.exp(logits - m)
    weights_ref[0] = e / jnp.sum(e, axis=-1, keepdims=True)

    ts = jnp.tanh(out[:, :sd] + bs_ref[...])
    part = jnp.sum(ts, axis=0, keepdims=True)

    @pl.when(s == 0)
    def _init():
        pooled_ref[0] = jnp.zeros_like(pooled_ref[0])

    pooled_ref[0] += part

    @pl.when(s == ns - 1)
    def _finish():
        pooled_ref[0] = pooled_ref[0] * (1.0 / (x_ref.shape[1] * ns))


def kernel(hidden, W_state, b_state, W_route, b_route):
    B, S, H = hidden.shape
    SD = W_state.shape[1]
    E = W_route.shape[1]
    BS = 1024
    ns = S // BS

    W_comb = jnp.concatenate([W_state, W_route], axis=1).astype(jnp.bfloat16)
    bs2 = b_state.reshape(1, SD)
    br2 = b_route.reshape(1, E)

    grid = (B, ns)
    out_shape = (
        jax.ShapeDtypeStruct((B, S, E), jnp.float32),
        jax.ShapeDtypeStruct((B, S, E), jnp.float32),
        jax.ShapeDtypeStruct((B, 1, SD), jnp.float32),
    )
    logits, weights, pooled = pl.pallas_call(
        functools.partial(_router_kernel, sd=SD, ne=E),
        grid=grid,
        in_specs=[
            pl.BlockSpec((1, BS, H), lambda b, s: (b, s, 0)),
            pl.BlockSpec((H, SD + E), lambda b, s: (0, 0)),
            pl.BlockSpec((1, SD), lambda b, s: (0, 0)),
            pl.BlockSpec((1, E), lambda b, s: (0, 0)),
        ],
        out_specs=(
            pl.BlockSpec((1, BS, E), lambda b, s: (b, s, 0)),
            pl.BlockSpec((1, BS, E), lambda b, s: (b, s, 0)),
            pl.BlockSpec((1, 1, SD), lambda b, s: (b, 0, 0)),
        ),
        out_shape=out_shape,
        compiler_params=pltpu.CompilerParams(
            dimension_semantics=("parallel", "arbitrary"),
        ),
    )(hidden, W_comb, bs2, br2)
    return (logits, weights, pooled)


# speedup vs baseline: 1.8623x; 1.8623x over previous
import jax
import jax.numpy as jnp
from jax.experimental import pallas as pl
from jax.experimental.pallas import tpu as pltpu


def _router_kernel(x_ref, ws_ref, bs_ref, wr_ref, br_ref,
                   logits_ref, weights_ref, pooled_ref):
    s = pl.program_id(1)
    ns = pl.num_programs(1)
    x = x_ref[0]

    logits = jnp.dot(x, wr_ref[...], preferred_element_type=jnp.float32)
    logits = logits + br_ref[...]
    logits_ref[0] = logits
    m = jnp.max(logits, axis=-1, keepdims=True)
    e = jnp.exp(logits - m)
    weights_ref[0] = e / jnp.sum(e, axis=-1, keepdims=True)

    ts = jnp.tanh(jnp.dot(x, ws_ref[...], preferred_element_type=jnp.float32)
                  + bs_ref[...])
    part = jnp.sum(ts, axis=0, keepdims=True)

    @pl.when(s == 0)
    def _init():
        pooled_ref[0] = jnp.zeros_like(pooled_ref[0])

    pooled_ref[0] += part

    @pl.when(s == ns - 1)
    def _finish():
        pooled_ref[0] = pooled_ref[0] * (1.0 / (x_ref.shape[1] * ns))


def kernel(hidden, W_state, b_state, W_route, b_route):
    B, S, H = hidden.shape
    SD = W_state.shape[1]
    E = W_route.shape[1]
    BS = 1024
    ns = S // BS

    bs2 = b_state.reshape(1, SD)
    br2 = b_route.reshape(1, E)

    grid = (B, ns)
    out_shape = (
        jax.ShapeDtypeStruct((B, S, E), jnp.float32),
        jax.ShapeDtypeStruct((B, S, E), jnp.float32),
        jax.ShapeDtypeStruct((B, 1, SD), jnp.float32),
    )
    logits, weights, pooled = pl.pallas_call(
        _router_kernel,
        grid=grid,
        in_specs=[
            pl.BlockSpec((1, BS, H), lambda b, s: (b, s, 0)),
            pl.BlockSpec((H, SD), lambda b, s: (0, 0)),
            pl.BlockSpec((1, SD), lambda b, s: (0, 0)),
            pl.BlockSpec((H, E), lambda b, s: (0, 0)),
            pl.BlockSpec((1, E), lambda b, s: (0, 0)),
        ],
        out_specs=(
            pl.BlockSpec((1, BS, E), lambda b, s: (b, s, 0)),
            pl.BlockSpec((1, BS, E), lambda b, s: (b, s, 0)),
            pl.BlockSpec((1, 1, SD), lambda b, s: (b, 0, 0)),
        ),
        out_shape=out_shape,
        compiler_params=pltpu.CompilerParams(
            dimension_semantics=("parallel", "arbitrary"),
        ),
    )(hidden, W_state, bs2, W_route, br2)
    return (logits, weights, pooled)


# speedup vs baseline: 2.1365x; 1.1472x over previous
import jax
import jax.numpy as jnp
from jax.experimental import pallas as pl
from jax.experimental.pallas import tpu as pltpu


def _router_kernel(x_ref, ws_ref, bs_ref, wr_ref, br_ref,
                   logits_ref, weights_ref, pooled_ref):
    s = pl.program_id(1)
    ns = pl.num_programs(1)
    x = x_ref[0]

    logits = jnp.dot(x, wr_ref[...], preferred_element_type=jnp.float32)
    logits = logits + br_ref[...]
    logits_ref[0] = logits
    m = jnp.max(logits, axis=-1, keepdims=True)
    e = jnp.exp(logits - m)
    weights_ref[0] = e / jnp.sum(e, axis=-1, keepdims=True)

    ts = jnp.tanh(jnp.dot(x, ws_ref[...], preferred_element_type=jnp.float32)
                  + bs_ref[...])
    part = jnp.sum(ts, axis=0, keepdims=True)

    @pl.when(s == 0)
    def _init():
        pooled_ref[0] = jnp.zeros_like(pooled_ref[0])

    pooled_ref[0] += part

    @pl.when(s == ns - 1)
    def _finish():
        pooled_ref[0] = pooled_ref[0] * (1.0 / (x_ref.shape[1] * ns))


def kernel(hidden, W_state, b_state, W_route, b_route):
    B, S, H = hidden.shape
    SD = W_state.shape[1]
    E = W_route.shape[1]
    BS = 2048
    ns = S // BS

    bs2 = b_state.reshape(1, SD)
    br2 = b_route.reshape(1, E)

    grid = (B, ns)
    out_shape = (
        jax.ShapeDtypeStruct((B, S, E), jnp.float32),
        jax.ShapeDtypeStruct((B, S, E), jnp.float32),
        jax.ShapeDtypeStruct((B, 1, SD), jnp.float32),
    )
    logits, weights, pooled = pl.pallas_call(
        _router_kernel,
        grid=grid,
        in_specs=[
            pl.BlockSpec((1, BS, H), lambda b, s: (b, s, 0)),
            pl.BlockSpec((H, SD), lambda b, s: (0, 0)),
            pl.BlockSpec((1, SD), lambda b, s: (0, 0)),
            pl.BlockSpec((H, E), lambda b, s: (0, 0)),
            pl.BlockSpec((1, E), lambda b, s: (0, 0)),
        ],
        out_specs=(
            pl.BlockSpec((1, BS, E), lambda b, s: (b, s, 0)),
            pl.BlockSpec((1, BS, E), lambda b, s: (b, s, 0)),
            pl.BlockSpec((1, 1, SD), lambda b, s: (b, 0, 0)),
        ),
        out_shape=out_shape,
        compiler_params=pltpu.CompilerParams(
            dimension_semantics=("parallel", "arbitrary"),
        ),
    )(hidden, W_state, bs2, W_route, br2)
    return (logits, weights, pooled)


# speedup vs baseline: 2.2061x; 1.0326x over previous
import jax
import jax.numpy as jnp
from jax.experimental import pallas as pl
from jax.experimental.pallas import tpu as pltpu


def _router_kernel(x_ref, ws_ref, bs_ref, wr_ref, br_ref,
                   logits_ref, weights_ref, pooled_ref):
    s = pl.program_id(1)
    ns = pl.num_programs(1)
    x = x_ref[0]

    logits = jnp.dot(x, wr_ref[...], preferred_element_type=jnp.float32)
    logits = logits + br_ref[...]
    logits_ref[0] = logits
    m = jnp.max(logits, axis=-1, keepdims=True)
    e = jnp.exp(logits - m)
    weights_ref[0] = e / jnp.sum(e, axis=-1, keepdims=True)

    ts = jnp.tanh(jnp.dot(x, ws_ref[...], preferred_element_type=jnp.float32)
                  + bs_ref[...])
    part = jnp.sum(ts, axis=0, keepdims=True)

    @pl.when(s == 0)
    def _init():
        pooled_ref[0] = jnp.zeros_like(pooled_ref[0])

    pooled_ref[0] += part

    @pl.when(s == ns - 1)
    def _finish():
        pooled_ref[0] = pooled_ref[0] * (1.0 / (x_ref.shape[1] * ns))


def kernel(hidden, W_state, b_state, W_route, b_route):
    B, S, H = hidden.shape
    SD = W_state.shape[1]
    E = W_route.shape[1]
    BS = 4096
    ns = S // BS

    bs2 = b_state.reshape(1, SD)
    br2 = b_route.reshape(1, E)

    grid = (B, ns)
    out_shape = (
        jax.ShapeDtypeStruct((B, S, E), jnp.float32),
        jax.ShapeDtypeStruct((B, S, E), jnp.float32),
        jax.ShapeDtypeStruct((B, 1, SD), jnp.float32),
    )
    logits, weights, pooled = pl.pallas_call(
        _router_kernel,
        grid=grid,
        in_specs=[
            pl.BlockSpec((1, BS, H), lambda b, s: (b, s, 0)),
            pl.BlockSpec((H, SD), lambda b, s: (0, 0)),
            pl.BlockSpec((1, SD), lambda b, s: (0, 0)),
            pl.BlockSpec((H, E), lambda b, s: (0, 0)),
            pl.BlockSpec((1, E), lambda b, s: (0, 0)),
        ],
        out_specs=(
            pl.BlockSpec((1, BS, E), lambda b, s: (b, s, 0)),
            pl.BlockSpec((1, BS, E), lambda b, s: (b, s, 0)),
            pl.BlockSpec((1, 1, SD), lambda b, s: (b, 0, 0)),
        ),
        out_shape=out_shape,
        compiler_params=pltpu.CompilerParams(
            dimension_semantics=("parallel", "arbitrary"),
        ),
    )(hidden, W_state, bs2, W_route, br2)
    return (logits, weights, pooled)
